# baseline (speedup 1.0000x reference)
import jax
import jax.numpy as jnp
from jax.experimental import pallas as pl

_INV_PI = 0.31830988618379067
_PI_A = 3.140625
_PI_B = 0.0009676535897932025


def _sincos(x):
    kf = jnp.round(x * _INV_PI)
    r = (x - kf * _PI_A) - kf * _PI_B
    r2 = r * r
    sp = r * (1.0 + r2 * (-1.6666667e-01 + r2 * (8.3333338e-03 + r2 * (
        -1.9841270e-04 + r2 * (2.7557319e-06 + r2 * -2.5052108e-08)))))
    cp = 1.0 + r2 * (-0.5 + r2 * (4.1666668e-02 + r2 * (-1.3888889e-03 + r2 * (
        2.4801587e-05 + r2 * (-2.7557319e-07 + r2 * 2.0876757e-09)))))
    sgn = 1.0 - 2.0 * (kf.astype(jnp.int32) & 1).astype(jnp.float32)
    return sp * sgn, cp * sgn

_B, _N, _L, _D = 4, 2048, 512, 2
_LATENT = 128
_NH = 128
_AD = 64
_H = 4
_NOUT = 3
_K = 16
_TQ = 512
_P = _TQ * _K


def _enf_kernel(x_ref, pT_ref, tab_ref,
                q_w1_ref, q_w2_ref, q_b2_ref, q_w3_ref, q_b3_ref,
                v_w1_ref, v_w2_ref, v_b2_ref, v_w3_ref, v_b3_ref,
                Wk_ref, bk_ref, Wv_ref, bv_ref,
                vm_w1_ref, vm_b1_ref, vm_w2_ref, vm_b2_ref,
                Wo_ref, bo_ref, o_ref):
    f32 = jnp.float32
    xb = x_ref[0]
    pT = pT_ref[0]
    tab = tab_ref[0]

    x0 = xb[:, 0:1]
    x1 = xb[:, 1:2]
    p0 = pT[0:1, :]
    p1 = pT[1:2, :]
    d0 = x0 - p0
    d1 = x1 - p1
    dist = d0 * d0 + d1 * d1

    lane_l = jax.lax.broadcasted_iota(jnp.int32, (_TQ, _L), 1)
    idx_cols = []
    dcur = dist
    for _ in range(_K):
        m = jnp.min(dcur, axis=1, keepdims=True)
        cand = jnp.where(dcur == m, lane_l, _L)
        idx_k = jnp.min(cand, axis=1, keepdims=True)
        idx_cols.append(idx_k)
        dcur = jnp.where(lane_l == idx_k, jnp.inf, dcur)
    idx = jnp.concatenate(idx_cols, axis=1)

    bf16 = jnp.bfloat16

    def hi(a):
        return a.astype(bf16).astype(f32)

    sub_p = jax.lax.broadcasted_iota(jnp.int32, (_P, _TQ), 0)
    lane_tq = jax.lax.broadcasted_iota(jnp.int32, (_P, _TQ), 1)
    oh_q = (sub_p // _K == lane_tq).astype(bf16)
    x0h = hi(x0)
    x0m = hi(x0 - x0h)
    x1h = hi(x1)
    x1m = hi(x1 - x1h)
    idx_lo = (idx & 255).astype(f32)
    idx_hi = (idx >> 8).astype(f32)
    small = jnp.concatenate(
        [x0h, x0m, x0 - x0h - x0m, x1h, x1m, x1 - x1h - x1m,
         idx_lo, idx_hi], axis=1)
    gsm = jnp.dot(oh_q, small.astype(bf16), preferred_element_type=f32)
    x0p = gsm[:, 0:1] + gsm[:, 1:2] + gsm[:, 2:3]
    x1p = gsm[:, 3:4] + gsm[:, 4:5] + gsm[:, 5:6]
    slotv = gsm[:, 6:6 + _K] + 256.0 * gsm[:, 6 + _K:6 + 2 * _K]
    ksel = (jax.lax.broadcasted_iota(jnp.int32, (_P, _K), 1)
            == jax.lax.broadcasted_iota(jnp.int32, (_P, _K), 0) % _K)
    idx_pair = jnp.sum(jnp.where(ksel, slotv, 0.0), axis=1, keepdims=True)

    lane_lp = jax.lax.broadcasted_iota(jnp.int32, (_P, _L), 1)
    onehot = (lane_lp == idx_pair.astype(jnp.int32)).astype(bf16)
    gp_cols = tab[:, _LATENT:_LATENT + 3]
    gph = hi(gp_cols)
    gpm = hi(gp_cols - gph)
    gpl = gp_cols - gph - gpm
    tab_b = jnp.concatenate(
        [tab[:, :_LATENT].astype(bf16), gph.astype(bf16),
         gpm.astype(bf16), gpl.astype(bf16)], axis=1)
    gtab = jnp.dot(onehot, tab_b, preferred_element_type=f32)
    c_g = gtab[:, :_LATENT]
    gn = (gtab[:, _LATENT:_LATENT + 1] + gtab[:, _LATENT + 3:_LATENT + 4]
          + gtab[:, _LATENT + 6:_LATENT + 7])
    p0g = (gtab[:, _LATENT + 1:_LATENT + 2] + gtab[:, _LATENT + 4:_LATENT + 5]
           + gtab[:, _LATENT + 7:_LATENT + 8])
    p1g = (gtab[:, _LATENT + 2:_LATENT + 3] + gtab[:, _LATENT + 5:_LATENT + 6]
           + gtab[:, _LATENT + 8:_LATENT + 9])

    bi0 = x0p - p0g
    bi1 = x1p - p1g
    zx = bi0 * bi0 + bi1 * bi1

    def bf(a):
        return a.astype(jnp.bfloat16)

    def rnd(a):
        return a.astype(jnp.bfloat16).astype(f32)

    def bdot(a, b):
        return jnp.dot(bf(a), bf(b), preferred_element_type=f32)

    sbi0, _ = _sincos(bi0)
    sbi1, _ = _sincos(bi1)

    def pos_emb(w1, w2, b2):
        e0 = rnd(jnp.pi * (bi0 + 1.0))
        e1 = rnd(jnp.pi * (bi1 + 1.0))
        emb = e0 * rnd(w1[0:1, :]) + e1 * rnd(w1[1:2, :])
        s_emb, c_emb = _sincos(emb)
        return (rnd(sbi0) * rnd(w2[0:1, :])
                + rnd(sbi1) * rnd(w2[1:2, :])
                + bdot(s_emb, w2[2:66, :])
                + bdot(c_emb, w2[66:130, :])
                + b2)

    pe_q = pos_emb(q_w1_ref[...], q_w2_ref[...], q_b2_ref[...])
    pe_v = pos_emb(v_w1_ref[...], v_w2_ref[...], v_b2_ref[...])
    q = bdot(pe_q, q_w3_ref[...]) + q_b3_ref[...]
    pv = bdot(pe_v, v_w3_ref[...]) + v_b3_ref[...]
    b_v = pv[:, :_NH]
    g_v = pv[:, _NH:]

    k_g = bdot(c_g, Wk_ref[...]) + bk_ref[...]
    v0 = bdot(c_g, Wv_ref[...]) + bv_ref[...]
    v1 = v0 * (1.0 + b_v) + g_v
    h1 = jax.nn.gelu(bdot(v1, vm_w1_ref[...]) + vm_b1_ref[...])
    v2 = bdot(h1, vm_w2_ref[...]) + vm_b2_ref[...]

    qk = q * k_g
    hs = jnp.concatenate(
        [jnp.sum(qk[:, h * _AD:(h + 1) * _AD], axis=1, keepdims=True)
         for h in range(_H)], axis=1)
    logits = hs - (1.0 / (gn * gn)) * zx

    l3 = logits.reshape(_TQ, _K, _H)
    m3 = jnp.max(l3, axis=1, keepdims=True)
    e3 = jnp.exp(l3 - m3)
    att3 = e3 / jnp.sum(e3, axis=1, keepdims=True)
    att = att3.reshape(_P, _H)

    ev = (jax.lax.broadcasted_iota(jnp.int32, (_H, _H * _NH), 0)
          == jax.lax.broadcasted_iota(jnp.int32, (_H, _H * _NH), 1) // _NH
          ).astype(bf16)
    att_h = hi(att)
    attf = (jnp.dot(att_h.astype(bf16), ev, preferred_element_type=f32)
            + jnp.dot((att - att_h).astype(bf16), ev,
                      preferred_element_type=f32))
    wv = (v2 * attf).reshape(_TQ, _K, _H * _NH)
    y = jnp.sum(wv, axis=1)

    o_ref[0] = bdot(y, Wo_ref[...]) + bo_ref[...]


def kernel(x, p, c, g, q_pe_w1, q_pe_w2, q_pe_b2, q_w3, q_b3,
           v_pe_w1, v_pe_w2, v_pe_b2, v_w3, v_b3, Wk, bk, Wv, bv,
           vm_w1, vm_b1, vm_w2, vm_b2, Wo, bo):
    pT = jnp.swapaxes(p, 1, 2)
    tab = jnp.concatenate([c, g, p], axis=2)

    def r(b):
        return b.reshape(1, -1)

    def im_x(b, n):
        return (b, n, 0)

    def im_rep3(b, n):
        return (b, 0, 0)

    def im_w(b, n):
        return (0, 0)

    return pl.pallas_call(
        _enf_kernel,
        grid=(_B, _N // _TQ),
        in_specs=[
            pl.BlockSpec((1, _TQ, _D), im_x),
            pl.BlockSpec((1, _D, _L), im_rep3),
            pl.BlockSpec((1, _L, _LATENT + 3), im_rep3),
            pl.BlockSpec((_D, _NH // 2), im_w),
            pl.BlockSpec((_D + _NH, _NH), im_w),
            pl.BlockSpec((1, _NH), im_w),
            pl.BlockSpec((_NH, _H * _AD), im_w),
            pl.BlockSpec((1, _H * _AD), im_w),
            pl.BlockSpec((_D, _NH // 2), im_w),
            pl.BlockSpec((_D + _NH, _NH), im_w),
            pl.BlockSpec((1, _NH), im_w),
            pl.BlockSpec((_NH, 2 * _NH), im_w),
            pl.BlockSpec((1, 2 * _NH), im_w),
            pl.BlockSpec((_LATENT, _H * _AD), im_w),
            pl.BlockSpec((1, _H * _AD), im_w),
            pl.BlockSpec((_LATENT, _NH), im_w),
            pl.BlockSpec((1, _NH), im_w),
            pl.BlockSpec((_NH, _NH), im_w),
            pl.BlockSpec((1, _NH), im_w),
            pl.BlockSpec((_NH, _H * _NH), im_w),
            pl.BlockSpec((1, _H * _NH), im_w),
            pl.BlockSpec((_H * _NH, _NOUT), im_w),
            pl.BlockSpec((1, _NOUT), im_w),
        ],
        out_specs=pl.BlockSpec((1, _TQ, _NOUT), im_x),
        out_shape=jax.ShapeDtypeStruct((_B, _N, _NOUT), jnp.float32),
    )(x, pT, tab, q_pe_w1, q_pe_w2, r(q_pe_b2), q_w3, r(q_b3),
      v_pe_w1, v_pe_w2, r(v_pe_b2), v_w3, r(v_b3),
      Wk, r(bk), Wv, r(bv), vm_w1, r(vm_b1), vm_w2, r(vm_b2), Wo, r(bo))


# speedup vs baseline: 1.0367x; 1.0367x over previous
import jax
import jax.numpy as jnp
from jax.experimental import pallas as pl

_INV_PI = 0.31830988618379067
_PI_A = 3.140625
_PI_B = 0.0009676535897932025


def _sincos(x):
    kf = jnp.round(x * _INV_PI)
    r = (x - kf * _PI_A) - kf * _PI_B
    r2 = r * r
    sp = r * (1.0 + r2 * (-1.6666667e-01 + r2 * (8.3333338e-03 + r2 * (
        -1.9841270e-04 + r2 * (2.7557319e-06 + r2 * -2.5052108e-08)))))
    cp = 1.0 + r2 * (-0.5 + r2 * (4.1666668e-02 + r2 * (-1.3888889e-03 + r2 * (
        2.4801587e-05 + r2 * (-2.7557319e-07 + r2 * 2.0876757e-09)))))
    sgn = 1.0 - 2.0 * (kf.astype(jnp.int32) & 1).astype(jnp.float32)
    return sp * sgn, cp * sgn

_B, _N, _L, _D = 4, 2048, 512, 2
_LATENT = 128
_NH = 128
_AD = 64
_H = 4
_NOUT = 3
_K = 16
_TQ = 128
_P = _TQ * _K


def _enf_kernel(x_ref, pT_ref, tab_ref,
                q_w1_ref, q_w2_ref, q_b2_ref, q_w3_ref, q_b3_ref,
                v_w1_ref, v_w2_ref, v_b2_ref, v_w3_ref, v_b3_ref,
                Wk_ref, bk_ref, Wv_ref, bv_ref,
                vm_w1_ref, vm_b1_ref, vm_w2_ref, vm_b2_ref,
                Wo_ref, bo_ref, o_ref):
    f32 = jnp.float32
    xb = x_ref[0]
    pT = pT_ref[0]
    tab = tab_ref[0]

    x0 = xb[:, 0:1]
    x1 = xb[:, 1:2]
    p0 = pT[0:1, :]
    p1 = pT[1:2, :]
    d0 = x0 - p0
    d1 = x1 - p1
    dist = d0 * d0 + d1 * d1

    lane_l = jax.lax.broadcasted_iota(jnp.int32, (_TQ, _L), 1)
    idx_cols = []
    dcur = dist
    for _ in range(_K):
        m = jnp.min(dcur, axis=1, keepdims=True)
        cand = jnp.where(dcur == m, lane_l, _L)
        idx_k = jnp.min(cand, axis=1, keepdims=True)
        idx_cols.append(idx_k)
        dcur = jnp.where(lane_l == idx_k, jnp.inf, dcur)
    idx = jnp.concatenate(idx_cols, axis=1)

    bf16 = jnp.bfloat16

    def hi(a):
        return a.astype(bf16).astype(f32)

    sub_p = jax.lax.broadcasted_iota(jnp.int32, (_P, _TQ), 0)
    lane_tq = jax.lax.broadcasted_iota(jnp.int32, (_P, _TQ), 1)
    oh_q = (sub_p // _K == lane_tq).astype(bf16)
    x0h = hi(x0)
    x0m = hi(x0 - x0h)
    x1h = hi(x1)
    x1m = hi(x1 - x1h)
    idx_lo = (idx & 255).astype(f32)
    idx_hi = (idx >> 8).astype(f32)
    small = jnp.concatenate(
        [x0h, x0m, x0 - x0h - x0m, x1h, x1m, x1 - x1h - x1m,
         idx_lo, idx_hi], axis=1)
    gsm = jnp.dot(oh_q, small.astype(bf16), preferred_element_type=f32)
    x0p = gsm[:, 0:1] + gsm[:, 1:2] + gsm[:, 2:3]
    x1p = gsm[:, 3:4] + gsm[:, 4:5] + gsm[:, 5:6]
    slotv = gsm[:, 6:6 + _K] + 256.0 * gsm[:, 6 + _K:6 + 2 * _K]
    ksel = (jax.lax.broadcasted_iota(jnp.int32, (_P, _K), 1)
            == jax.lax.broadcasted_iota(jnp.int32, (_P, _K), 0) % _K)
    idx_pair = jnp.sum(jnp.where(ksel, slotv, 0.0), axis=1, keepdims=True)

    lane_lp = jax.lax.broadcasted_iota(jnp.int32, (_P, _L), 1)
    onehot = (lane_lp == idx_pair.astype(jnp.int32)).astype(bf16)
    gp_cols = tab[:, _LATENT:_LATENT + 3]
    gph = hi(gp_cols)
    gpm = hi(gp_cols - gph)
    gpl = gp_cols - gph - gpm
    tab_b = jnp.concatenate(
        [tab[:, :_LATENT].astype(bf16), gph.astype(bf16),
         gpm.astype(bf16), gpl.astype(bf16)], axis=1)
    gtab = jnp.dot(onehot, tab_b, preferred_element_type=f32)
    c_g = gtab[:, :_LATENT]
    gn = (gtab[:, _LATENT:_LATENT + 1] + gtab[:, _LATENT + 3:_LATENT + 4]
          + gtab[:, _LATENT + 6:_LATENT + 7])
    p0g = (gtab[:, _LATENT + 1:_LATENT + 2] + gtab[:, _LATENT + 4:_LATENT + 5]
           + gtab[:, _LATENT + 7:_LATENT + 8])
    p1g = (gtab[:, _LATENT + 2:_LATENT + 3] + gtab[:, _LATENT + 5:_LATENT + 6]
           + gtab[:, _LATENT + 8:_LATENT + 9])

    bi0 = x0p - p0g
    bi1 = x1p - p1g
    zx = bi0 * bi0 + bi1 * bi1

    def bf(a):
        return a.astype(jnp.bfloat16)

    def rnd(a):
        return a.astype(jnp.bfloat16).astype(f32)

    def bdot(a, b):
        return jnp.dot(bf(a), bf(b), preferred_element_type=f32)

    sbi0, _ = _sincos(bi0)
    sbi1, _ = _sincos(bi1)

    def pos_emb(w1, w2, b2):
        e0 = rnd(jnp.pi * (bi0 + 1.0))
        e1 = rnd(jnp.pi * (bi1 + 1.0))
        emb = e0 * rnd(w1[0:1, :]) + e1 * rnd(w1[1:2, :])
        s_emb, c_emb = _sincos(emb)
        return (rnd(sbi0) * rnd(w2[0:1, :])
                + rnd(sbi1) * rnd(w2[1:2, :])
                + bdot(s_emb, w2[2:66, :])
                + bdot(c_emb, w2[66:130, :])
                + b2)

    pe_q = pos_emb(q_w1_ref[...], q_w2_ref[...], q_b2_ref[...])
    pe_v = pos_emb(v_w1_ref[...], v_w2_ref[...], v_b2_ref[...])
    q = bdot(pe_q, q_w3_ref[...]) + q_b3_ref[...]
    pv = bdot(pe_v, v_w3_ref[...]) + v_b3_ref[...]
    b_v = pv[:, :_NH]
    g_v = pv[:, _NH:]

    k_g = bdot(c_g, Wk_ref[...]) + bk_ref[...]
    v0 = bdot(c_g, Wv_ref[...]) + bv_ref[...]
    v1 = v0 * (1.0 + b_v) + g_v
    h1 = jax.nn.gelu(bdot(v1, vm_w1_ref[...]) + vm_b1_ref[...])
    v2 = bdot(h1, vm_w2_ref[...]) + vm_b2_ref[...]

    qk = q * k_g
    hs = jnp.concatenate(
        [jnp.sum(qk[:, h * _AD:(h + 1) * _AD], axis=1, keepdims=True)
         for h in range(_H)], axis=1)
    logits = hs - (1.0 / (gn * gn)) * zx

    l3 = logits.reshape(_TQ, _K, _H)
    m3 = jnp.max(l3, axis=1, keepdims=True)
    e3 = jnp.exp(l3 - m3)
    att3 = e3 / jnp.sum(e3, axis=1, keepdims=True)
    att = att3.reshape(_P, _H)

    ev = (jax.lax.broadcasted_iota(jnp.int32, (_H, _H * _NH), 0)
          == jax.lax.broadcasted_iota(jnp.int32, (_H, _H * _NH), 1) // _NH
          ).astype(bf16)
    att_h = hi(att)
    attf = (jnp.dot(att_h.astype(bf16), ev, preferred_element_type=f32)
            + jnp.dot((att - att_h).astype(bf16), ev,
                      preferred_element_type=f32))
    wv = (v2 * attf).reshape(_TQ, _K, _H * _NH)
    y = jnp.sum(wv, axis=1)

    o_ref[0] = bdot(y, Wo_ref[...]) + bo_ref[...]


def kernel(x, p, c, g, q_pe_w1, q_pe_w2, q_pe_b2, q_w3, q_b3,
           v_pe_w1, v_pe_w2, v_pe_b2, v_w3, v_b3, Wk, bk, Wv, bv,
           vm_w1, vm_b1, vm_w2, vm_b2, Wo, bo):
    pT = jnp.swapaxes(p, 1, 2)
    tab = jnp.concatenate([c, g, p], axis=2)

    def r(b):
        return b.reshape(1, -1)

    def im_x(b, n):
        return (b, n, 0)

    def im_rep3(b, n):
        return (b, 0, 0)

    def im_w(b, n):
        return (0, 0)

    return pl.pallas_call(
        _enf_kernel,
        grid=(_B, _N // _TQ),
        in_specs=[
            pl.BlockSpec((1, _TQ, _D), im_x),
            pl.BlockSpec((1, _D, _L), im_rep3),
            pl.BlockSpec((1, _L, _LATENT + 3), im_rep3),
            pl.BlockSpec((_D, _NH // 2), im_w),
            pl.BlockSpec((_D + _NH, _NH), im_w),
            pl.BlockSpec((1, _NH), im_w),
            pl.BlockSpec((_NH, _H * _AD), im_w),
            pl.BlockSpec((1, _H * _AD), im_w),
            pl.BlockSpec((_D, _NH // 2), im_w),
            pl.BlockSpec((_D + _NH, _NH), im_w),
            pl.BlockSpec((1, _NH), im_w),
            pl.BlockSpec((_NH, 2 * _NH), im_w),
            pl.BlockSpec((1, 2 * _NH), im_w),
            pl.BlockSpec((_LATENT, _H * _AD), im_w),
            pl.BlockSpec((1, _H * _AD), im_w),
            pl.BlockSpec((_LATENT, _NH), im_w),
            pl.BlockSpec((1, _NH), im_w),
            pl.BlockSpec((_NH, _NH), im_w),
            pl.BlockSpec((1, _NH), im_w),
            pl.BlockSpec((_NH, _H * _NH), im_w),
            pl.BlockSpec((1, _H * _NH), im_w),
            pl.BlockSpec((_H * _NH, _NOUT), im_w),
            pl.BlockSpec((1, _NOUT), im_w),
        ],
        out_specs=pl.BlockSpec((1, _TQ, _NOUT), im_x),
        out_shape=jax.ShapeDtypeStruct((_B, _N, _NOUT), jnp.float32),
    )(x, pT, tab, q_pe_w1, q_pe_w2, r(q_pe_b2), q_w3, r(q_b3),
      v_pe_w1, v_pe_w2, r(v_pe_b2), v_w3, r(v_b3),
      Wk, r(bk), Wv, r(bv), vm_w1, r(vm_b1), vm_w2, r(vm_b2), Wo, r(bo))


# speedup vs baseline: 1.1727x; 1.1312x over previous
import jax
import jax.numpy as jnp
from jax.experimental import pallas as pl

_INV_PI = 0.31830988618379067
_PI_A = 3.140625
_PI_B = 0.0009676535897932025


def _sincos(x):
    kf = jnp.round(x * _INV_PI)
    r = (x - kf * _PI_A) - kf * _PI_B
    r2 = r * r
    sp = r * (1.0 + r2 * (-1.6666667e-01 + r2 * (8.3333338e-03 + r2 * (
        -1.9841270e-04 + r2 * (2.7557319e-06 + r2 * -2.5052108e-08)))))
    cp = 1.0 + r2 * (-0.5 + r2 * (4.1666668e-02 + r2 * (-1.3888889e-03 + r2 * (
        2.4801587e-05 + r2 * (-2.7557319e-07 + r2 * 2.0876757e-09)))))
    sgn = 1.0 - 2.0 * (kf.astype(jnp.int32) & 1).astype(jnp.float32)
    return sp * sgn, cp * sgn

_B, _N, _L, _D = 4, 2048, 512, 2
_LATENT = 128
_NH = 128
_AD = 64
_H = 4
_NOUT = 3
_K = 16
_TQ = 256
_P = _TQ * _K


def _enf_kernel(x_ref, pT_ref, tab_ref,
                q_w1_ref, q_w2_ref, q_b2_ref, q_w3_ref, q_b3_ref,
                v_w1_ref, v_w2_ref, v_b2_ref, v_w3_ref, v_b3_ref,
                Wk_ref, bk_ref, Wv_ref, bv_ref,
                vm_w1_ref, vm_b1_ref, vm_w2_ref, vm_b2_ref,
                Wo_ref, bo_ref, o_ref):
    f32 = jnp.float32
    xb = x_ref[0]
    pT = pT_ref[0]
    tab = tab_ref[0]

    x0 = xb[:, 0:1]
    x1 = xb[:, 1:2]
    p0 = pT[0:1, :]
    p1 = pT[1:2, :]
    d0 = x0 - p0
    d1 = x1 - p1
    dist = d0 * d0 + d1 * d1

    lane_l = jax.lax.broadcasted_iota(jnp.int32, (_TQ, _L), 1)
    idx_cols = []
    dcur = dist
    for _ in range(_K):
        idx_k = jnp.argmin(dcur, axis=1, keepdims=True).astype(jnp.int32)
        idx_cols.append(idx_k)
        dcur = jnp.where(lane_l == idx_k, jnp.inf, dcur)
    idx = jnp.concatenate(idx_cols, axis=1)

    bf16 = jnp.bfloat16

    def hi(a):
        return a.astype(bf16).astype(f32)

    sub_p = jax.lax.broadcasted_iota(jnp.int32, (_P, _TQ), 0)
    lane_tq = jax.lax.broadcasted_iota(jnp.int32, (_P, _TQ), 1)
    oh_q = (sub_p // _K == lane_tq).astype(bf16)
    x0h = hi(x0)
    x0m = hi(x0 - x0h)
    x1h = hi(x1)
    x1m = hi(x1 - x1h)
    idx_lo = (idx & 255).astype(f32)
    idx_hi = (idx >> 8).astype(f32)
    small = jnp.concatenate(
        [x0h, x0m, x0 - x0h - x0m, x1h, x1m, x1 - x1h - x1m,
         idx_lo, idx_hi], axis=1)
    gsm = jnp.dot(oh_q, small.astype(bf16), preferred_element_type=f32)
    x0p = gsm[:, 0:1] + gsm[:, 1:2] + gsm[:, 2:3]
    x1p = gsm[:, 3:4] + gsm[:, 4:5] + gsm[:, 5:6]
    slotv = gsm[:, 6:6 + _K] + 256.0 * gsm[:, 6 + _K:6 + 2 * _K]
    ksel = (jax.lax.broadcasted_iota(jnp.int32, (_P, _K), 1)
            == jax.lax.broadcasted_iota(jnp.int32, (_P, _K), 0) % _K)
    idx_pair = jnp.sum(jnp.where(ksel, slotv, 0.0), axis=1, keepdims=True)

    lane_lp = jax.lax.broadcasted_iota(jnp.int32, (_P, _L), 1)
    onehot = (lane_lp == idx_pair.astype(jnp.int32)).astype(bf16)
    gp_cols = tab[:, _LATENT:_LATENT + 3]
    gph = hi(gp_cols)
    gpm = hi(gp_cols - gph)
    gpl = gp_cols - gph - gpm
    tab_b = jnp.concatenate(
        [tab[:, :_LATENT].astype(bf16), gph.astype(bf16),
         gpm.astype(bf16), gpl.astype(bf16)], axis=1)
    gtab = jnp.dot(onehot, tab_b, preferred_element_type=f32)
    c_g = gtab[:, :_LATENT]
    gn = (gtab[:, _LATENT:_LATENT + 1] + gtab[:, _LATENT + 3:_LATENT + 4]
          + gtab[:, _LATENT + 6:_LATENT + 7])
    p0g = (gtab[:, _LATENT + 1:_LATENT + 2] + gtab[:, _LATENT + 4:_LATENT + 5]
           + gtab[:, _LATENT + 7:_LATENT + 8])
    p1g = (gtab[:, _LATENT + 2:_LATENT + 3] + gtab[:, _LATENT + 5:_LATENT + 6]
           + gtab[:, _LATENT + 8:_LATENT + 9])

    bi0 = x0p - p0g
    bi1 = x1p - p1g
    zx = bi0 * bi0 + bi1 * bi1

    def bf(a):
        return a.astype(jnp.bfloat16)

    def rnd(a):
        return a.astype(jnp.bfloat16).astype(f32)

    def bdot(a, b):
        return jnp.dot(bf(a), bf(b), preferred_element_type=f32)

    sbi0, _ = _sincos(bi0)
    sbi1, _ = _sincos(bi1)

    def pos_emb(w1, w2, b2):
        e0 = rnd(jnp.pi * (bi0 + 1.0))
        e1 = rnd(jnp.pi * (bi1 + 1.0))
        emb = e0 * rnd(w1[0:1, :]) + e1 * rnd(w1[1:2, :])
        s_emb, c_emb = _sincos(emb)
        return (rnd(sbi0) * rnd(w2[0:1, :])
                + rnd(sbi1) * rnd(w2[1:2, :])
                + bdot(s_emb, w2[2:66, :])
                + bdot(c_emb, w2[66:130, :])
                + b2)

    pe_q = pos_emb(q_w1_ref[...], q_w2_ref[...], q_b2_ref[...])
    pe_v = pos_emb(v_w1_ref[...], v_w2_ref[...], v_b2_ref[...])
    q = bdot(pe_q, q_w3_ref[...]) + q_b3_ref[...]
    pv = bdot(pe_v, v_w3_ref[...]) + v_b3_ref[...]
    b_v = pv[:, :_NH]
    g_v = pv[:, _NH:]

    k_g = bdot(c_g, Wk_ref[...]) + bk_ref[...]
    v0 = bdot(c_g, Wv_ref[...]) + bv_ref[...]
    v1 = v0 * (1.0 + b_v) + g_v
    h1 = jax.nn.gelu(bdot(v1, vm_w1_ref[...]) + vm_b1_ref[...])
    v2 = bdot(h1, vm_w2_ref[...]) + vm_b2_ref[...]

    qk = q * k_g
    hs = jnp.concatenate(
        [jnp.sum(qk[:, h * _AD:(h + 1) * _AD], axis=1, keepdims=True)
         for h in range(_H)], axis=1)
    logits = hs - (1.0 / (gn * gn)) * zx

    l3 = logits.reshape(_TQ, _K, _H)
    m3 = jnp.max(l3, axis=1, keepdims=True)
    e3 = jnp.exp(l3 - m3)
    att3 = e3 / jnp.sum(e3, axis=1, keepdims=True)
    att = att3.reshape(_P, _H)

    ev = (jax.lax.broadcasted_iota(jnp.int32, (_H, _H * _NH), 0)
          == jax.lax.broadcasted_iota(jnp.int32, (_H, _H * _NH), 1) // _NH
          ).astype(bf16)
    att_h = hi(att)
    attf = (jnp.dot(att_h.astype(bf16), ev, preferred_element_type=f32)
            + jnp.dot((att - att_h).astype(bf16), ev,
                      preferred_element_type=f32))
    wv = (v2 * attf).reshape(_TQ, _K, _H * _NH)
    y = jnp.sum(wv, axis=1)

    o_ref[0] = bdot(y, Wo_ref[...]) + bo_ref[...]


def kernel(x, p, c, g, q_pe_w1, q_pe_w2, q_pe_b2, q_w3, q_b3,
           v_pe_w1, v_pe_w2, v_pe_b2, v_w3, v_b3, Wk, bk, Wv, bv,
           vm_w1, vm_b1, vm_w2, vm_b2, Wo, bo):
    pT = jnp.swapaxes(p, 1, 2)
    tab = jnp.concatenate([c, g, p], axis=2)

    def r(b):
        return b.reshape(1, -1)

    def im_x(b, n):
        return (b, n, 0)

    def im_rep3(b, n):
        return (b, 0, 0)

    def im_w(b, n):
        return (0, 0)

    return pl.pallas_call(
        _enf_kernel,
        grid=(_B, _N // _TQ),
        in_specs=[
            pl.BlockSpec((1, _TQ, _D), im_x),
            pl.BlockSpec((1, _D, _L), im_rep3),
            pl.BlockSpec((1, _L, _LATENT + 3), im_rep3),
            pl.BlockSpec((_D, _NH // 2), im_w),
            pl.BlockSpec((_D + _NH, _NH), im_w),
            pl.BlockSpec((1, _NH), im_w),
            pl.BlockSpec((_NH, _H * _AD), im_w),
            pl.BlockSpec((1, _H * _AD), im_w),
            pl.BlockSpec((_D, _NH // 2), im_w),
            pl.BlockSpec((_D + _NH, _NH), im_w),
            pl.BlockSpec((1, _NH), im_w),
            pl.BlockSpec((_NH, 2 * _NH), im_w),
            pl.BlockSpec((1, 2 * _NH), im_w),
            pl.BlockSpec((_LATENT, _H * _AD), im_w),
            pl.BlockSpec((1, _H * _AD), im_w),
            pl.BlockSpec((_LATENT, _NH), im_w),
            pl.BlockSpec((1, _NH), im_w),
            pl.BlockSpec((_NH, _NH), im_w),
            pl.BlockSpec((1, _NH), im_w),
            pl.BlockSpec((_NH, _H * _NH), im_w),
            pl.BlockSpec((1, _H * _NH), im_w),
            pl.BlockSpec((_H * _NH, _NOUT), im_w),
            pl.BlockSpec((1, _NOUT), im_w),
        ],
        out_specs=pl.BlockSpec((1, _TQ, _NOUT), im_x),
        out_shape=jax.ShapeDtypeStruct((_B, _N, _NOUT), jnp.float32),
    )(x, pT, tab, q_pe_w1, q_pe_w2, r(q_pe_b2), q_w3, r(q_b3),
      v_pe_w1, v_pe_w2, r(v_pe_b2), v_w3, r(v_b3),
      Wk, r(bk), Wv, r(bv), vm_w1, r(vm_b1), vm_w2, r(vm_b2), Wo, r(bo))
